# Initial kernel scaffold; baseline (speedup 1.0000x reference)
import jax
import jax.numpy as jnp
from jax.experimental import pallas as pl


def kernel(x, edge_index, edge_weight, conv1_W0, conv1_W1, conv1_b, convs_W0, convs_W1, convs_b, fc1_W, fc1_b, fc2_W, fc2_b, fc3_W, fc3_b):
    raise NotImplementedError("write your pallas kernel here")


# speedup vs baseline: 1.0309x; 1.0309x over previous
import functools

import jax
import jax.numpy as jnp
from jax import lax
from jax.experimental import pallas as pl
from jax.experimental.pallas import tpu as pltpu


def _tc_first_body(x_ref, w0_ref, w1_ref, b_ref, a_ref, b_out_ref):
    x = x_ref[...]
    a_ref[...] = jnp.dot(x, w0_ref[...], preferred_element_type=jnp.float32) + b_ref[...]
    b_out_ref[...] = jnp.dot(x, w1_ref[...], preferred_element_type=jnp.float32)


def _tc_mid_body(p_ref, aprev_ref, w0_ref, w1_ref, b_ref, a_ref, b_out_ref):
    h = aprev_ref[...] + jnp.sum(p_ref[...], axis=0)
    h = jnp.maximum(h, 0.0)
    a_ref[...] = jnp.dot(h, w0_ref[...], preferred_element_type=jnp.float32) + b_ref[...]
    b_out_ref[...] = jnp.dot(h, w1_ref[...], preferred_element_type=jnp.float32)


def _tc_fin_body(p_ref, aprev_ref, g_ref):
    g_ref[...] = aprev_ref[...] + jnp.sum(p_ref[...], axis=0)


def _tc_matmuls(p, a_prev, w0, w1, b, first, x=None):
    n = a_prev.shape[0] if a_prev is not None else x.shape[0]
    nf = w0.shape[1]
    out_shape = [jax.ShapeDtypeStruct((n, nf), jnp.float32)] * 2
    if first:
        return pl.pallas_call(
            _tc_first_body,
            out_shape=out_shape,
        )(x, w0, w1, b.reshape(1, -1))
    return pl.pallas_call(
        _tc_mid_body,
        out_shape=out_shape,
    )(p, a_prev, w0, w1, b.reshape(1, -1))


def _tc_finalize(p, a_prev):
    return pl.pallas_call(
        _tc_fin_body,
        out_shape=jax.ShapeDtypeStruct(a_prev.shape, jnp.float32),
    )(p, a_prev)


def _fc_body(flat_ref, w1_ref, b1_ref, w2_ref, b2_ref, w3_ref, b3_ref,
             out_ref, acc_ref):
    k = pl.program_id(0)

    @pl.when(k == 0)
    def _():
        acc_ref[...] = jnp.zeros_like(acc_ref)

    acc_ref[...] += jnp.dot(flat_ref[...], w1_ref[...],
                            preferred_element_type=jnp.float32)

    @pl.when(k == pl.num_programs(0) - 1)
    def _():
        y = acc_ref[...] + b1_ref[...]
        y = jnp.dot(y, w2_ref[...], preferred_element_type=jnp.float32) + b2_ref[...]
        y = jnp.dot(y, w3_ref[...], preferred_element_type=jnp.float32) + b3_ref[...]
        out_ref[...] = y


def _fc_head(flat, fc1_w, fc1_b, fc2_w, fc2_b, fc3_w, fc3_b):
    ktot = fc1_w.shape[0]
    bk = 3200
    steps = ktot // bk
    return pl.pallas_call(
        _fc_body,
        grid=(steps,),
        in_specs=[
            pl.BlockSpec((1, bk), lambda k: (0, k)),
            pl.BlockSpec((bk, 256), lambda k: (k, 0)),
            pl.BlockSpec((1, 256), lambda k: (0, 0)),
            pl.BlockSpec((256, 128), lambda k: (0, 0)),
            pl.BlockSpec((1, 128), lambda k: (0, 0)),
            pl.BlockSpec((128, 2), lambda k: (0, 0)),
            pl.BlockSpec((1, 2), lambda k: (0, 0)),
        ],
        out_specs=pl.BlockSpec((1, 2), lambda k: (0, 0)),
        out_shape=jax.ShapeDtypeStruct((1, 2), jnp.float32),
        scratch_shapes=[pltpu.VMEM((1, 256), jnp.float32)],
    )(flat, fc1_w, fc1_b.reshape(1, -1), fc2_w, fc2_b.reshape(1, -1),
      fc3_w, fc3_b.reshape(1, -1))


def _edge_norm(row, col, w, n):
    deg = jnp.zeros((n,), w.dtype).at[row].add(w)
    dinv = jnp.where(deg > 0.0, lax.rsqrt(deg), 0.0)
    return -(dinv[row] * w * dinv[col])


def _spmm(row, col, neg_w, bmat):
    n = bmat.shape[0]
    tx1 = jnp.zeros_like(bmat).at[col].add(bmat[row] * neg_w[:, None])
    return tx1[None]


def kernel(x, edge_index, edge_weight, conv1_W0, conv1_W1, conv1_b,
           convs_W0, convs_W1, convs_b, fc1_W, fc1_b, fc2_W, fc2_b,
           fc3_W, fc3_b):
    n = x.shape[0]
    row, col = edge_index[0], edge_index[1]
    neg_w = _edge_norm(row, col, edge_weight, n)

    a, b = _tc_matmuls(None, None, conv1_W0, conv1_W1, conv1_b, True, x=x)
    for i in range(5):
        p = _spmm(row, col, neg_w, b)
        a, b = _tc_matmuls(p, a, convs_W0[i], convs_W1[i], convs_b[i], False)
    p = _spmm(row, col, neg_w, b)
    g = _tc_finalize(p, a)
    flat = g.reshape(1, -1)
    return _fc_head(flat, fc1_W, fc1_b, fc2_W, fc2_b, fc3_W, fc3_b)


# speedup vs baseline: 5.0526x; 4.9010x over previous
import functools

import jax
import jax.numpy as jnp
from jax import lax
from jax.experimental import pallas as pl
from jax.experimental.pallas import tpu as pltpu
from jax.experimental.pallas import tpu_sc as plsc

_N = 10000
_NPAD = 10240
_E = 640000
_NF = 32
_CW = 80
_NT = 16
_NW = 32


def _tc_layer_body(h_ref, p_ref, w0_ref, w1_ref, b_ref, out_ref, *, relu):
    tx1 = p_ref[0] + p_ref[1]
    m = (jnp.dot(h_ref[...], w0_ref[...], preferred_element_type=jnp.float32)
         + jnp.dot(tx1, w1_ref[...], preferred_element_type=jnp.float32)
         + b_ref[...])
    out_ref[...] = jnp.maximum(m, 0.0) if relu else m


def _tc_layer(h, p, w0, w1, b, relu):
    nf = w0.shape[1]
    return pl.pallas_call(
        functools.partial(_tc_layer_body, relu=relu),
        out_shape=jax.ShapeDtypeStruct((h.shape[0], nf), jnp.float32),
    )(h, p, w0, w1, b.reshape(1, -1))


def _fc_body(flat_ref, w1_ref, b1_ref, w2_ref, b2_ref, w3_ref, b3_ref,
             out_ref, acc_ref):
    k = pl.program_id(0)

    @pl.when(k == 0)
    def _():
        acc_ref[...] = jnp.zeros_like(acc_ref)

    acc_ref[...] += jnp.dot(flat_ref[...], w1_ref[...],
                            preferred_element_type=jnp.float32)

    @pl.when(k == pl.num_programs(0) - 1)
    def _():
        y = acc_ref[...] + b1_ref[...]
        y = jnp.dot(y, w2_ref[...], preferred_element_type=jnp.float32) + b2_ref[...]
        y = jnp.dot(y, w3_ref[...], preferred_element_type=jnp.float32) + b3_ref[...]
        out_ref[...] = y


def _fc_head(flat, fc1_w, fc1_b, fc2_w, fc2_b, fc3_w, fc3_b):
    ktot = fc1_w.shape[0]
    bk = 3200
    steps = ktot // bk
    return pl.pallas_call(
        _fc_body,
        grid=(steps,),
        in_specs=[
            pl.BlockSpec((1, bk), lambda k: (0, k)),
            pl.BlockSpec((bk, 256), lambda k: (k, 0)),
            pl.BlockSpec((1, 256), lambda k: (0, 0)),
            pl.BlockSpec((256, 128), lambda k: (0, 0)),
            pl.BlockSpec((1, 128), lambda k: (0, 0)),
            pl.BlockSpec((128, 2), lambda k: (0, 0)),
            pl.BlockSpec((1, 2), lambda k: (0, 0)),
        ],
        out_specs=pl.BlockSpec((1, 2), lambda k: (0, 0)),
        out_shape=jax.ShapeDtypeStruct((1, 2), jnp.float32),
        scratch_shapes=[pltpu.VMEM((1, 256), jnp.float32)],
    )(flat, fc1_w, fc1_b.reshape(1, -1), fc2_w, fc2_b.reshape(1, -1),
      fc3_w, fc3_b.reshape(1, -1))


def _rsqrt_nr(x):
    i = lax.bitcast_convert_type(x, jnp.int32)
    i = jnp.int32(0x5F3759DF) - jnp.right_shift(i, 1)
    y = lax.bitcast_convert_type(i, jnp.float32)
    for _ in range(3):
        y = y * (1.5 - 0.5 * x * y * y)
    return jnp.where(x > 0.0, y, 0.0)


def _sc_mesh():
    return plsc.VectorSubcoreMesh(core_axis_name="c", subcore_axis_name="s")


_SL = _NPAD // _NT
_NBLK = _E // (8 * _CW)


@functools.partial(
    pl.kernel,
    out_type=jax.ShapeDtypeStruct((_NBLK, 8, _CW), jnp.float32),
    mesh=_sc_mesh(),
    compiler_params=pltpu.CompilerParams(needs_layout_passes=False, use_tc_tiling_on_sc=False),
    scratch_types=[
        pltpu.VMEM_SHARED((_NPAD,), jnp.float32),
        pltpu.VMEM((8, _CW), jnp.int32),
        pltpu.VMEM((8, _CW), jnp.int32),
        pltpu.VMEM((8, _CW), jnp.float32),
        pltpu.VMEM((8, _CW), jnp.float32),
        pltpu.VMEM((_NPAD,), jnp.float32),
        pltpu.VMEM((_SL,), jnp.float32),
    ],
)
def _sc_edge_norm(row3, col3, w3, negw_hbm, deg_sh,
                  row_b, col_b, w_b, out_b, dinv_t, sl_b):
    c = lax.axis_index("c")
    s = lax.axis_index("s")
    wid = c * _NT + s
    zero = jnp.zeros((16,), jnp.float32)
    for j in range(_SL // 16):
        sl_b[pl.ds(j * 16, 16)] = zero
    pltpu.sync_copy(sl_b, deg_sh.at[pl.ds(s * _SL, _SL)])
    plsc.subcore_barrier()

    nblk_t = jnp.where(s < 8, _NBLK // _NT + 1, _NBLK // _NT)
    t0 = s * (_NBLK // _NT) + jnp.minimum(s, 8)

    def deg_chunk(i, _):
        blk = t0 + i
        pltpu.sync_copy(row3.at[blk], row_b)
        pltpu.sync_copy(w3.at[blk], w_b)
        for j in range(8):
            pltpu.sync_copy(w_b.at[j], deg_sh.at[row_b.at[j]], add=True)
        return ()

    lax.fori_loop(0, nblk_t, deg_chunk, ())
    plsc.subcore_barrier()

    pltpu.sync_copy(deg_sh.at[pl.ds(s * _SL, _SL)], sl_b)
    for j in range(_SL // 16):
        sl_b[pl.ds(j * 16, 16)] = _rsqrt_nr(sl_b[pl.ds(j * 16, 16)])
    plsc.subcore_barrier()
    pltpu.sync_copy(sl_b, deg_sh.at[pl.ds(s * _SL, _SL)])
    plsc.subcore_barrier()
    pltpu.sync_copy(deg_sh, dinv_t)

    nblk_w = jnp.where(wid < 8, _NBLK // _NW + 1, _NBLK // _NW)
    w0 = wid * (_NBLK // _NW) + jnp.minimum(wid, 8)

    def nw_chunk(i, _):
        blk = w0 + i
        pltpu.sync_copy(row3.at[blk], row_b)
        pltpu.sync_copy(col3.at[blk], col_b)
        pltpu.sync_copy(w3.at[blk], w_b)
        for j in range(8):
            for g in range(_CW // 16):
                r16 = row_b[j, pl.ds(g * 16, 16)]
                c16 = col_b[j, pl.ds(g * 16, 16)]
                wv = w_b[j, pl.ds(g * 16, 16)]
                g1 = plsc.load_gather(dinv_t, [r16])
                g2 = plsc.load_gather(dinv_t, [c16])
                out_b[j, pl.ds(g * 16, 16)] = -(g1 * wv * g2)
        pltpu.sync_copy(out_b, negw_hbm.at[blk])
        return ()

    lax.fori_loop(0, nblk_w, nw_chunk, ())


@functools.lru_cache(maxsize=None)
def _make_sc_spmm(F):
    @functools.partial(
        pl.kernel,
        out_type=jax.ShapeDtypeStruct((2, _NPAD, F), jnp.float32),
        mesh=_sc_mesh(),
        compiler_params=pltpu.CompilerParams(needs_layout_passes=False, use_tc_tiling_on_sc=False),
        scratch_types=[
            pltpu.VMEM_SHARED((_NPAD, F), jnp.float32),
            pltpu.VMEM((8, _CW), jnp.int32),
            pltpu.VMEM((8, _CW), jnp.int32),
            pltpu.VMEM((8, _CW), jnp.float32),
            pltpu.VMEM((8, _CW, F), jnp.float32),
            pltpu.VMEM((_CW, F), jnp.float32),
        ],
    )
    def _sc_spmm(row3, col3, negw, hmat, out_hbm, acc_sh,
                 row_b, col_b, nw_b, rows_b, z_b):
        c = lax.axis_index("c")
        s = lax.axis_index("s")
        wid = c * _NT + s
        zero = jnp.zeros((16,), jnp.float32)
        for i in range(_CW):
            for f0 in range(0, F, 16):
                z_b[i, pl.ds(f0, 16)] = zero
        for k in range(_SL // _CW):
            pltpu.sync_copy(z_b, acc_sh.at[pl.ds(s * _SL + k * _CW, _CW)])
        plsc.subcore_barrier()

        nblk_w = jnp.where(wid < 8, _NBLK // _NW + 1, _NBLK // _NW)
        w0 = wid * (_NBLK // _NW) + jnp.minimum(wid, 8)
        iota = lax.iota(jnp.int32, 16)

        def chunk(i, _):
            blk = w0 + i
            pltpu.sync_copy(row3.at[blk], row_b)
            pltpu.sync_copy(col3.at[blk], col_b)
            pltpu.sync_copy(negw.at[blk], nw_b)

            def sub(j, _):
                pltpu.sync_copy(hmat.at[row_b.at[j]], rows_b.at[j])
                j16 = jnp.full((16,), j, jnp.int32)
                for g in range(_CW // 16):
                    wv = nw_b[j, pl.ds(g * 16, 16)]
                    k16 = iota + (g * 16)
                    for f in range(F):
                        f16 = jnp.full((16,), f, jnp.int32)
                        v = plsc.load_gather(rows_b, [j16, k16, f16])
                        plsc.store_scatter(rows_b, [j16, k16, f16], v * wv)
                pltpu.sync_copy(rows_b.at[j], acc_sh.at[col_b.at[j]], add=True)
                return ()

            lax.fori_loop(0, 8, sub, ())
            return ()

        lax.fori_loop(0, nblk_w, chunk, ())
        plsc.subcore_barrier()
        pltpu.sync_copy(acc_sh.at[pl.ds(s * _SL, _SL)],
                        out_hbm.at[c].at[pl.ds(s * _SL, _SL)])

    return _sc_spmm


def _edge_norm(row3, col3, w3):
    return _sc_edge_norm(row3, col3, w3)


def _spmm(row3, col3, neg_w, hmat):
    return _make_sc_spmm(hmat.shape[1])(row3, col3, neg_w, hmat)


def kernel(x, edge_index, edge_weight, conv1_W0, conv1_W1, conv1_b,
           convs_W0, convs_W1, convs_b, fc1_W, fc1_b, fc2_W, fc2_b,
           fc3_W, fc3_b):
    row, col = edge_index[0], edge_index[1]
    row3 = row.reshape(_NBLK, 8, _CW)
    col3 = col.reshape(_NBLK, 8, _CW)
    w3 = edge_weight.reshape(_NBLK, 8, _CW)
    neg_w = _edge_norm(row3, col3, w3)

    t = x.shape[1]
    tpad = 64
    xp = jnp.pad(x, ((0, _NPAD - _N), (0, tpad - t)))
    w0p = jnp.pad(conv1_W0, ((0, tpad - t), (0, 0)))
    w1p = jnp.pad(conv1_W1, ((0, tpad - t), (0, 0)))
    p = _spmm(row3, col3, neg_w, xp)
    h = _tc_layer(xp, p, w0p, w1p, conv1_b, True)
    for i in range(5):
        p = _spmm(row3, col3, neg_w, h)
        h = _tc_layer(h, p, convs_W0[i], convs_W1[i], convs_b[i], i < 4)
    flat = h[:_N].reshape(1, -1)
    return _fc_head(flat, fc1_W, fc1_b, fc2_W, fc2_b, fc3_W, fc3_b)


# speedup vs baseline: 6.0870x; 1.2047x over previous
import functools

import jax
import jax.numpy as jnp
from jax import lax
from jax.experimental import pallas as pl
from jax.experimental.pallas import tpu as pltpu
from jax.experimental.pallas import tpu_sc as plsc

_N = 10000
_NPAD = 10240
_E = 640000
_NF = 32
_CW = 80
_NT = 16
_NW = 32


def _tc_layer_body(h_ref, p_ref, w0_ref, w1_ref, b_ref, out_ref, *, relu):
    tx1 = p_ref[0] + p_ref[1]
    m = (jnp.dot(h_ref[...], w0_ref[...], preferred_element_type=jnp.float32)
         + jnp.dot(tx1, w1_ref[...], preferred_element_type=jnp.float32)
         + b_ref[...])
    out_ref[...] = jnp.maximum(m, 0.0) if relu else m


def _tc_layer(h, p, w0, w1, b, relu):
    nf = w0.shape[1]
    return pl.pallas_call(
        functools.partial(_tc_layer_body, relu=relu),
        out_shape=jax.ShapeDtypeStruct((h.shape[0], nf), jnp.float32),
    )(h, p, w0, w1, b.reshape(1, -1))


def _fc_body(flat_ref, w1_ref, b1_ref, w2_ref, b2_ref, w3_ref, b3_ref,
             out_ref, acc_ref):
    k = pl.program_id(0)

    @pl.when(k == 0)
    def _():
        acc_ref[...] = jnp.zeros_like(acc_ref)

    acc_ref[...] += jnp.dot(flat_ref[...], w1_ref[...],
                            preferred_element_type=jnp.float32)

    @pl.when(k == pl.num_programs(0) - 1)
    def _():
        y = acc_ref[...] + b1_ref[...]
        y = jnp.dot(y, w2_ref[...], preferred_element_type=jnp.float32) + b2_ref[...]
        y = jnp.dot(y, w3_ref[...], preferred_element_type=jnp.float32) + b3_ref[...]
        out_ref[...] = y


def _fc_head(flat, fc1_w, fc1_b, fc2_w, fc2_b, fc3_w, fc3_b):
    ktot = fc1_w.shape[0]
    bk = 3200
    steps = ktot // bk
    return pl.pallas_call(
        _fc_body,
        grid=(steps,),
        in_specs=[
            pl.BlockSpec((1, bk), lambda k: (0, k)),
            pl.BlockSpec((bk, 256), lambda k: (k, 0)),
            pl.BlockSpec((1, 256), lambda k: (0, 0)),
            pl.BlockSpec((256, 128), lambda k: (0, 0)),
            pl.BlockSpec((1, 128), lambda k: (0, 0)),
            pl.BlockSpec((128, 2), lambda k: (0, 0)),
            pl.BlockSpec((1, 2), lambda k: (0, 0)),
        ],
        out_specs=pl.BlockSpec((1, 2), lambda k: (0, 0)),
        out_shape=jax.ShapeDtypeStruct((1, 2), jnp.float32),
        scratch_shapes=[pltpu.VMEM((1, 256), jnp.float32)],
    )(flat, fc1_w, fc1_b.reshape(1, -1), fc2_w, fc2_b.reshape(1, -1),
      fc3_w, fc3_b.reshape(1, -1))


def _rsqrt_nr(x):
    i = lax.bitcast_convert_type(x, jnp.int32)
    i = jnp.int32(0x5F3759DF) - jnp.right_shift(i, 1)
    y = lax.bitcast_convert_type(i, jnp.float32)
    for _ in range(3):
        y = y * (1.5 - 0.5 * x * y * y)
    return jnp.where(x > 0.0, y, 0.0)


def _sc_mesh():
    return plsc.VectorSubcoreMesh(core_axis_name="c", subcore_axis_name="s")


_SL = _NPAD // _NT
_NBLK = _E // (8 * _CW)


@functools.partial(
    pl.kernel,
    out_type=jax.ShapeDtypeStruct((_NBLK, 8, _CW), jnp.float32),
    mesh=_sc_mesh(),
    compiler_params=pltpu.CompilerParams(needs_layout_passes=False, use_tc_tiling_on_sc=False),
    scratch_types=[
        pltpu.VMEM_SHARED((_NPAD,), jnp.float32),
        pltpu.VMEM((8, _CW), jnp.int32),
        pltpu.VMEM((8, _CW), jnp.int32),
        pltpu.VMEM((8, _CW), jnp.float32),
        pltpu.VMEM((8, _CW), jnp.float32),
        pltpu.VMEM((_NPAD,), jnp.float32),
        pltpu.VMEM((_SL,), jnp.float32),
    ],
)
def _sc_edge_norm(row3, col3, w3, negw_hbm, deg_sh,
                  row_b, col_b, w_b, out_b, dinv_t, sl_b):
    c = lax.axis_index("c")
    s = lax.axis_index("s")
    wid = c * _NT + s
    zero = jnp.zeros((16,), jnp.float32)
    for j in range(_SL // 16):
        sl_b[pl.ds(j * 16, 16)] = zero
    pltpu.sync_copy(sl_b, deg_sh.at[pl.ds(s * _SL, _SL)])
    plsc.subcore_barrier()

    nblk_t = jnp.where(s < 8, _NBLK // _NT + 1, _NBLK // _NT)
    t0 = s * (_NBLK // _NT) + jnp.minimum(s, 8)

    def deg_chunk(i, _):
        blk = t0 + i
        pltpu.sync_copy(row3.at[blk], row_b)
        pltpu.sync_copy(w3.at[blk], w_b)
        for j in range(8):
            pltpu.sync_copy(w_b.at[j], deg_sh.at[row_b.at[j]], add=True)
        return ()

    lax.fori_loop(0, nblk_t, deg_chunk, ())
    plsc.subcore_barrier()

    pltpu.sync_copy(deg_sh.at[pl.ds(s * _SL, _SL)], sl_b)
    for j in range(_SL // 16):
        sl_b[pl.ds(j * 16, 16)] = _rsqrt_nr(sl_b[pl.ds(j * 16, 16)])
    plsc.subcore_barrier()
    pltpu.sync_copy(sl_b, deg_sh.at[pl.ds(s * _SL, _SL)])
    plsc.subcore_barrier()
    pltpu.sync_copy(deg_sh, dinv_t)

    nblk_w = jnp.where(wid < 8, _NBLK // _NW + 1, _NBLK // _NW)
    w0 = wid * (_NBLK // _NW) + jnp.minimum(wid, 8)

    def nw_chunk(i, _):
        blk = w0 + i
        pltpu.sync_copy(row3.at[blk], row_b)
        pltpu.sync_copy(col3.at[blk], col_b)
        pltpu.sync_copy(w3.at[blk], w_b)
        for j in range(8):
            for g in range(_CW // 16):
                r16 = row_b[j, pl.ds(g * 16, 16)]
                c16 = col_b[j, pl.ds(g * 16, 16)]
                wv = w_b[j, pl.ds(g * 16, 16)]
                g1 = plsc.load_gather(dinv_t, [r16])
                g2 = plsc.load_gather(dinv_t, [c16])
                out_b[j, pl.ds(g * 16, 16)] = -(g1 * wv * g2)
        pltpu.sync_copy(out_b, negw_hbm.at[blk])
        return ()

    lax.fori_loop(0, nblk_w, nw_chunk, ())


@functools.lru_cache(maxsize=None)
def _make_sc_spmm(F):
    JB = 512 // F
    NB = _E // (JB * _CW)
    q, r = divmod(NB, _NW)

    @functools.partial(
        pl.kernel,
        out_type=jax.ShapeDtypeStruct((2, _NPAD, F), jnp.float32),
        mesh=_sc_mesh(),
        compiler_params=pltpu.CompilerParams(needs_layout_passes=False, use_tc_tiling_on_sc=False),
        scratch_types=[
            pltpu.VMEM_SHARED((_NPAD, F), jnp.float32),
            pltpu.VMEM((3, JB, _CW), jnp.int32),
            pltpu.VMEM((3, JB, _CW), jnp.int32),
            pltpu.VMEM((3, JB, _CW), jnp.float32),
            pltpu.VMEM((2, JB * _CW, F), jnp.float32),
            pltpu.VMEM((16, F), jnp.float32),
            pltpu.SemaphoreType.DMA,
            pltpu.SemaphoreType.DMA,
            pltpu.SemaphoreType.DMA,
        ],
    )
    def _sc_spmm(row3, col3, negw, hmat, out_hbm, acc_sh,
                 row_b, col_b, nw_b, rows_b, z_b, sem_i, sem_g, sem_s):
        c = lax.axis_index("c")
        s = lax.axis_index("s")
        wid = c * _NT + s
        zero = jnp.zeros((16,), jnp.float32)
        for i in range(16):
            for f0 in range(0, F, 16):
                z_b[i, pl.ds(f0, 16)] = zero
        for k in range(_SL // 16):
            pltpu.sync_copy(z_b, acc_sh.at[pl.ds(s * _SL + k * 16, 16)])
        plsc.subcore_barrier()

        nb = jnp.where(wid < r, q + 1, q)
        b0 = wid * q + jnp.minimum(wid, r)
        iota = lax.iota(jnp.int32, 16)

        def idx_load(t, slot):
            blk = b0 + t
            pltpu.async_copy(row3.at[blk], row_b.at[slot], sem_i)
            pltpu.async_copy(col3.at[blk], col_b.at[slot], sem_i)
            pltpu.async_copy(negw.at[blk], nw_b.at[slot], sem_i)

        def idx_wait(slot):
            pltpu.make_async_copy(row3.at[b0], row_b.at[slot], sem_i).wait()
            pltpu.make_async_copy(col3.at[b0], col_b.at[slot], sem_i).wait()
            pltpu.make_async_copy(negw.at[b0], nw_b.at[slot], sem_i).wait()

        def gather_start(islot, rslot):
            for j in range(JB):
                pltpu.async_copy(hmat.at[row_b.at[islot].at[j]],
                                 rows_b.at[rslot].at[pl.ds(j * _CW, _CW)], sem_g)

        def gather_wait(rslot):
            for j in range(JB):
                pltpu.make_async_copy(hmat.at[pl.ds(0, _CW)],
                                      rows_b.at[rslot].at[pl.ds(j * _CW, _CW)], sem_g).wait()

        def scatter_start(islot, rslot):
            for j in range(JB):
                pltpu.async_copy(rows_b.at[rslot].at[pl.ds(j * _CW, _CW)],
                                 acc_sh.at[col_b.at[islot].at[j]], sem_s,
                                 add=True)

        def scatter_wait(rslot):
            for j in range(JB):
                pltpu.make_async_copy(rows_b.at[rslot].at[pl.ds(j * _CW, _CW)],
                                      acc_sh.at[pl.ds(0, _CW)], sem_s).wait()

        idx_load(0, 0)
        idx_wait(0)
        gather_start(0, 0)

        @pl.when(nb > 1)
        def _():
            idx_load(1, 1)

        def step(t, _):
            cur3 = lax.rem(t, 3)
            cur2 = lax.rem(t, 2)
            nxt3 = lax.rem(t + 1, 3)
            nxt2 = lax.rem(t + 1, 2)

            @pl.when(t >= 1)
            def _():
                scatter_wait(nxt2)

            @pl.when(t + 2 < nb)
            def _():
                idx_load(t + 2, lax.rem(t + 2, 3))

            gather_wait(cur2)
            j16s = jnp.full((16,), cur2, jnp.int32)

            def scale_j(j, _):
                k16b = iota + j * _CW
                for g in range(_CW // 16):
                    wv = nw_b[cur3, j, pl.ds(g * 16, 16)]
                    k16 = k16b + g * 16
                    for f in range(F):
                        f16 = jnp.full((16,), f, jnp.int32)
                        v = plsc.load_gather(rows_b, [j16s, k16, f16])
                        plsc.store_scatter(rows_b, [j16s, k16, f16], v * wv)
                return ()

            lax.fori_loop(0, JB, scale_j, ())
            scatter_start(cur3, cur2)

            @pl.when(t + 1 < nb)
            def _():
                idx_wait(nxt3)
                gather_start(nxt3, nxt2)
            return ()

        lax.fori_loop(0, nb, step, ())
        scatter_wait(lax.rem(nb - 1, 2))
        plsc.subcore_barrier()
        pltpu.sync_copy(acc_sh.at[pl.ds(s * _SL, _SL)],
                        out_hbm.at[c].at[pl.ds(s * _SL, _SL)])

    return _sc_spmm


def _edge_norm(row3, col3, w3):
    return _sc_edge_norm(row3, col3, w3)


def _spmm(row, col, neg_w, hmat):
    F = hmat.shape[1]
    JB = 512 // F
    nb = _E // (JB * _CW)
    r3 = row.reshape(nb, JB, _CW)
    c3 = col.reshape(nb, JB, _CW)
    n3 = neg_w.reshape(nb, JB, _CW)
    return _make_sc_spmm(F)(r3, c3, n3, hmat)


def kernel(x, edge_index, edge_weight, conv1_W0, conv1_W1, conv1_b,
           convs_W0, convs_W1, convs_b, fc1_W, fc1_b, fc2_W, fc2_b,
           fc3_W, fc3_b):
    row, col = edge_index[0], edge_index[1]
    row3 = row.reshape(_NBLK, 8, _CW)
    col3 = col.reshape(_NBLK, 8, _CW)
    w3 = edge_weight.reshape(_NBLK, 8, _CW)
    neg_w = _edge_norm(row3, col3, w3)

    t = x.shape[1]
    tpad = 64
    xp = jnp.pad(x, ((0, _NPAD - _N), (0, tpad - t)))
    w0p = jnp.pad(conv1_W0, ((0, tpad - t), (0, 0)))
    w1p = jnp.pad(conv1_W1, ((0, tpad - t), (0, 0)))
    negw_flat = neg_w.reshape(-1)
    p = _spmm(row, col, negw_flat, xp)
    h = _tc_layer(xp, p, w0p, w1p, conv1_b, True)
    for i in range(5):
        p = _spmm(row, col, negw_flat, h)
        h = _tc_layer(h, p, convs_W0[i], convs_W1[i], convs_b[i], i < 4)
    flat = h[:_N].reshape(1, -1)
    return _fc_head(flat, fc1_W, fc1_b, fc2_W, fc2_b, fc3_W, fc3_b)


# speedup vs baseline: 17.8381x; 2.9305x over previous
import functools

import jax
import jax.numpy as jnp
from jax import lax
from jax.experimental import pallas as pl
from jax.experimental.pallas import tpu as pltpu
from jax.experimental.pallas import tpu_sc as plsc

_N = 10000
_NPAD = 10240
_E = 640000
_NF = 32
_CW = 80
_NT = 16
_NW = 32


def _tc_layer_body(h_ref, p_ref, w0_ref, w1_ref, b_ref, out_ref, *, relu):
    tx1 = p_ref[0] + p_ref[1]
    m = (jnp.dot(h_ref[...], w0_ref[...], preferred_element_type=jnp.float32)
         + jnp.dot(tx1, w1_ref[...], preferred_element_type=jnp.float32)
         + b_ref[...])
    out_ref[...] = jnp.maximum(m, 0.0) if relu else m


def _tc_layer(h, p, w0, w1, b, relu):
    nf = w0.shape[1]
    return pl.pallas_call(
        functools.partial(_tc_layer_body, relu=relu),
        out_shape=jax.ShapeDtypeStruct((h.shape[0], nf), jnp.float32),
    )(h, p, w0, w1, b.reshape(1, -1))


def _fc_body(flat_ref, w1_ref, b1_ref, w2_ref, b2_ref, w3_ref, b3_ref,
             out_ref, acc_ref):
    k = pl.program_id(0)

    @pl.when(k == 0)
    def _():
        acc_ref[...] = jnp.zeros_like(acc_ref)

    acc_ref[...] += jnp.dot(flat_ref[...], w1_ref[...],
                            preferred_element_type=jnp.float32)

    @pl.when(k == pl.num_programs(0) - 1)
    def _():
        y = acc_ref[...] + b1_ref[...]
        y = jnp.dot(y, w2_ref[...], preferred_element_type=jnp.float32) + b2_ref[...]
        y = jnp.dot(y, w3_ref[...], preferred_element_type=jnp.float32) + b3_ref[...]
        out_ref[...] = y


def _fc_head(flat, fc1_w, fc1_b, fc2_w, fc2_b, fc3_w, fc3_b):
    ktot = fc1_w.shape[0]
    bk = 3200
    steps = ktot // bk
    return pl.pallas_call(
        _fc_body,
        grid=(steps,),
        in_specs=[
            pl.BlockSpec((1, bk), lambda k: (0, k)),
            pl.BlockSpec((bk, 256), lambda k: (k, 0)),
            pl.BlockSpec((1, 256), lambda k: (0, 0)),
            pl.BlockSpec((256, 128), lambda k: (0, 0)),
            pl.BlockSpec((1, 128), lambda k: (0, 0)),
            pl.BlockSpec((128, 2), lambda k: (0, 0)),
            pl.BlockSpec((1, 2), lambda k: (0, 0)),
        ],
        out_specs=pl.BlockSpec((1, 2), lambda k: (0, 0)),
        out_shape=jax.ShapeDtypeStruct((1, 2), jnp.float32),
        scratch_shapes=[pltpu.VMEM((1, 256), jnp.float32)],
    )(flat, fc1_w, fc1_b.reshape(1, -1), fc2_w, fc2_b.reshape(1, -1),
      fc3_w, fc3_b.reshape(1, -1))


def _rsqrt_nr(x):
    i = lax.bitcast_convert_type(x, jnp.int32)
    i = jnp.int32(0x5F3759DF) - jnp.right_shift(i, 1)
    y = lax.bitcast_convert_type(i, jnp.float32)
    for _ in range(3):
        y = y * (1.5 - 0.5 * x * y * y)
    return jnp.where(x > 0.0, y, 0.0)


def _sc_mesh():
    return plsc.VectorSubcoreMesh(core_axis_name="c", subcore_axis_name="s")


_SL = _NPAD // _NT
_NBLK = _E // (8 * _CW)


@functools.partial(
    pl.kernel,
    out_type=jax.ShapeDtypeStruct((_NBLK, 8, _CW), jnp.float32),
    mesh=_sc_mesh(),
    compiler_params=pltpu.CompilerParams(needs_layout_passes=False, use_tc_tiling_on_sc=False),
    scratch_types=[
        pltpu.VMEM_SHARED((_NPAD,), jnp.float32),
        pltpu.VMEM((8, _CW), jnp.int32),
        pltpu.VMEM((8, _CW), jnp.int32),
        pltpu.VMEM((8, _CW), jnp.float32),
        pltpu.VMEM((8, _CW), jnp.float32),
        pltpu.VMEM((_NPAD,), jnp.float32),
        pltpu.VMEM((_SL,), jnp.float32),
    ],
)
def _sc_edge_norm(row3, col3, w3, negw_hbm, deg_sh,
                  row_b, col_b, w_b, out_b, dinv_t, sl_b):
    c = lax.axis_index("c")
    s = lax.axis_index("s")
    wid = c * _NT + s
    zero = jnp.zeros((16,), jnp.float32)
    for j in range(_SL // 16):
        sl_b[pl.ds(j * 16, 16)] = zero
    pltpu.sync_copy(sl_b, deg_sh.at[pl.ds(s * _SL, _SL)])
    plsc.subcore_barrier()

    nblk_t = jnp.where(s < 8, _NBLK // _NT + 1, _NBLK // _NT)
    t0 = s * (_NBLK // _NT) + jnp.minimum(s, 8)

    def deg_chunk(i, _):
        blk = t0 + i
        pltpu.sync_copy(row3.at[blk], row_b)
        pltpu.sync_copy(w3.at[blk], w_b)
        for j in range(8):
            pltpu.sync_copy(w_b.at[j], deg_sh.at[row_b.at[j]], add=True)
        return ()

    lax.fori_loop(0, nblk_t, deg_chunk, ())
    plsc.subcore_barrier()

    pltpu.sync_copy(deg_sh.at[pl.ds(s * _SL, _SL)], sl_b)
    for j in range(_SL // 16):
        sl_b[pl.ds(j * 16, 16)] = _rsqrt_nr(sl_b[pl.ds(j * 16, 16)])
    plsc.subcore_barrier()
    pltpu.sync_copy(sl_b, deg_sh.at[pl.ds(s * _SL, _SL)])
    plsc.subcore_barrier()
    pltpu.sync_copy(deg_sh, dinv_t)

    nblk_w = jnp.where(wid < 8, _NBLK // _NW + 1, _NBLK // _NW)
    w0 = wid * (_NBLK // _NW) + jnp.minimum(wid, 8)

    def nw_chunk(i, _):
        blk = w0 + i
        pltpu.sync_copy(row3.at[blk], row_b)
        pltpu.sync_copy(col3.at[blk], col_b)
        pltpu.sync_copy(w3.at[blk], w_b)
        for j in range(8):
            for g in range(_CW // 16):
                r16 = row_b[j, pl.ds(g * 16, 16)]
                c16 = col_b[j, pl.ds(g * 16, 16)]
                wv = w_b[j, pl.ds(g * 16, 16)]
                g1 = plsc.load_gather(dinv_t, [r16])
                g2 = plsc.load_gather(dinv_t, [c16])
                out_b[j, pl.ds(g * 16, 16)] = -(g1 * wv * g2)
        pltpu.sync_copy(out_b, negw_hbm.at[blk])
        return ()

    lax.fori_loop(0, nblk_w, nw_chunk, ())


@functools.lru_cache(maxsize=None)
def _make_sc_spmm(F):
    JB = 512 // F
    NB = _E // (JB * _CW)
    q, r = divmod(NB, _NW)

    @functools.partial(
        pl.kernel,
        out_type=jax.ShapeDtypeStruct((2, _NPAD, F), jnp.float32),
        mesh=_sc_mesh(),
        compiler_params=pltpu.CompilerParams(needs_layout_passes=False, use_tc_tiling_on_sc=False),
        scratch_types=[
            pltpu.VMEM_SHARED((_NPAD, F), jnp.float32),
            pltpu.VMEM((3, JB, _CW), jnp.int32),
            pltpu.VMEM((3, JB, _CW), jnp.int32),
            pltpu.VMEM((3, JB, _CW), jnp.float32),
            pltpu.VMEM((2, JB * _CW, F), jnp.float32),
            pltpu.VMEM((16, F), jnp.float32),
            pltpu.SemaphoreType.DMA,
            pltpu.SemaphoreType.DMA,
            pltpu.SemaphoreType.DMA,
        ],
    )
    def _sc_spmm(row3, col3, negw, hmat, out_hbm, acc_sh,
                 row_b, col_b, nw_b, rows_b, z_b, sem_i, sem_g, sem_s):
        c = lax.axis_index("c")
        s = lax.axis_index("s")
        wid = c * _NT + s
        zero = jnp.zeros((16,), jnp.float32)
        for i in range(16):
            for f0 in range(0, F, 16):
                z_b[i, pl.ds(f0, 16)] = zero
        for k in range(_SL // 16):
            pltpu.sync_copy(z_b, acc_sh.at[pl.ds(s * _SL + k * 16, 16)])
        plsc.subcore_barrier()

        nb = jnp.where(wid < r, q + 1, q)
        b0 = wid * q + jnp.minimum(wid, r)
        iota = lax.iota(jnp.int32, 16)

        def idx_load(t, slot):
            blk = b0 + t
            pltpu.async_copy(row3.at[blk], row_b.at[slot], sem_i)
            pltpu.async_copy(col3.at[blk], col_b.at[slot], sem_i)
            pltpu.async_copy(negw.at[blk], nw_b.at[slot], sem_i)

        def idx_wait(slot):
            pltpu.make_async_copy(row3.at[b0], row_b.at[slot], sem_i).wait()
            pltpu.make_async_copy(col3.at[b0], col_b.at[slot], sem_i).wait()
            pltpu.make_async_copy(negw.at[b0], nw_b.at[slot], sem_i).wait()

        def gather_start(islot, rslot):
            for j in range(JB):
                pltpu.async_copy(hmat.at[row_b.at[islot].at[j]],
                                 rows_b.at[rslot].at[pl.ds(j * _CW, _CW)], sem_g)

        def gather_wait(rslot):
            for j in range(JB):
                pltpu.make_async_copy(hmat.at[pl.ds(0, _CW)],
                                      rows_b.at[rslot].at[pl.ds(j * _CW, _CW)], sem_g).wait()

        def scatter_start(islot, rslot):
            for j in range(JB):
                pltpu.async_copy(rows_b.at[rslot].at[pl.ds(j * _CW, _CW)],
                                 acc_sh.at[col_b.at[islot].at[j]], sem_s,
                                 add=True)

        def scatter_wait(rslot):
            for j in range(JB):
                pltpu.make_async_copy(rows_b.at[rslot].at[pl.ds(j * _CW, _CW)],
                                      acc_sh.at[pl.ds(0, _CW)], sem_s).wait()

        idx_load(0, 0)
        idx_wait(0)
        gather_start(0, 0)

        @pl.when(nb > 1)
        def _():
            idx_load(1, 1)

        def step(t, _):
            cur3 = lax.rem(t, 3)
            cur2 = lax.rem(t, 2)
            nxt3 = lax.rem(t + 1, 3)
            nxt2 = lax.rem(t + 1, 2)

            @pl.when(t >= 1)
            def _():
                scatter_wait(nxt2)

            @pl.when(t + 2 < nb)
            def _():
                idx_load(t + 2, lax.rem(t + 2, 3))

            gather_wait(cur2)
            def scale_j(j, _):
                for g in range(_CW // 16):
                    wv = nw_b[cur3, j, pl.ds(g * 16, 16)]
                    for l in range(16):
                        e = j * _CW + g * 16 + l
                        ws = lax.gather(
                            wv, jnp.full((16, 1), l, jnp.int32),
                            lax.GatherDimensionNumbers(
                                offset_dims=(), collapsed_slice_dims=(0,),
                                start_index_map=(0,)),
                            (1,), mode=lax.GatherScatterMode.PROMISE_IN_BOUNDS)
                        for f0 in range(0, F, 16):
                            v = rows_b[cur2, e, pl.ds(f0, 16)]
                            rows_b[cur2, e, pl.ds(f0, 16)] = v * ws
                return ()

            lax.fori_loop(0, JB, scale_j, ())
            scatter_start(cur3, cur2)

            @pl.when(t + 1 < nb)
            def _():
                idx_wait(nxt3)
                gather_start(nxt3, nxt2)
            return ()

        lax.fori_loop(0, nb, step, ())
        scatter_wait(lax.rem(nb - 1, 2))
        plsc.subcore_barrier()
        pltpu.sync_copy(acc_sh.at[pl.ds(s * _SL, _SL)],
                        out_hbm.at[c].at[pl.ds(s * _SL, _SL)])

    return _sc_spmm


def _edge_norm(row3, col3, w3):
    return _sc_edge_norm(row3, col3, w3)


def _spmm(row, col, neg_w, hmat):
    F = hmat.shape[1]
    JB = 512 // F
    nb = _E // (JB * _CW)
    r3 = row.reshape(nb, JB, _CW)
    c3 = col.reshape(nb, JB, _CW)
    n3 = neg_w.reshape(nb, JB, _CW)
    return _make_sc_spmm(F)(r3, c3, n3, hmat)


def kernel(x, edge_index, edge_weight, conv1_W0, conv1_W1, conv1_b,
           convs_W0, convs_W1, convs_b, fc1_W, fc1_b, fc2_W, fc2_b,
           fc3_W, fc3_b):
    row, col = edge_index[0], edge_index[1]
    row3 = row.reshape(_NBLK, 8, _CW)
    col3 = col.reshape(_NBLK, 8, _CW)
    w3 = edge_weight.reshape(_NBLK, 8, _CW)
    neg_w = _edge_norm(row3, col3, w3)

    t = x.shape[1]
    tpad = 64
    xp = jnp.pad(x, ((0, _NPAD - _N), (0, tpad - t)))
    w0p = jnp.pad(conv1_W0, ((0, tpad - t), (0, 0)))
    w1p = jnp.pad(conv1_W1, ((0, tpad - t), (0, 0)))
    negw_flat = neg_w.reshape(-1)
    p = _spmm(row, col, negw_flat, xp)
    h = _tc_layer(xp, p, w0p, w1p, conv1_b, True)
    for i in range(5):
        p = _spmm(row, col, negw_flat, h)
        h = _tc_layer(h, p, convs_W0[i], convs_W1[i], convs_b[i], i < 4)
    flat = h[:_N].reshape(1, -1)
    return _fc_head(flat, fc1_W, fc1_b, fc2_W, fc2_b, fc3_W, fc3_b)


# speedup vs baseline: 17.9412x; 1.0058x over previous
import functools

import jax
import jax.numpy as jnp
from jax import lax
from jax.experimental import pallas as pl
from jax.experimental.pallas import tpu as pltpu
from jax.experimental.pallas import tpu_sc as plsc

_N = 10000
_NPAD = 10240
_E = 640000
_NF = 32
_CW = 80
_NT = 16
_NW = 32


def _tc_layer_body(h_ref, p_ref, w0_ref, w1_ref, b_ref, out_ref, *, relu):
    tx1 = p_ref[0] + p_ref[1]
    m = (jnp.dot(h_ref[...], w0_ref[...], preferred_element_type=jnp.float32)
         + jnp.dot(tx1, w1_ref[...], preferred_element_type=jnp.float32)
         + b_ref[...])
    out_ref[...] = jnp.maximum(m, 0.0) if relu else m


def _tc_layer(h, p, w0, w1, b, relu):
    nf = w0.shape[1]
    return pl.pallas_call(
        functools.partial(_tc_layer_body, relu=relu),
        out_shape=jax.ShapeDtypeStruct((h.shape[0], nf), jnp.float32),
    )(h, p, w0, w1, b.reshape(1, -1))


def _fc_body(flat_ref, w1_ref, b1_ref, w2_ref, b2_ref, w3_ref, b3_ref,
             out_ref, acc_ref):
    k = pl.program_id(0)

    @pl.when(k == 0)
    def _():
        acc_ref[...] = jnp.zeros_like(acc_ref)

    acc_ref[...] += jnp.dot(flat_ref[...], w1_ref[...],
                            preferred_element_type=jnp.float32)

    @pl.when(k == pl.num_programs(0) - 1)
    def _():
        y = acc_ref[...] + b1_ref[...]
        y = jnp.dot(y, w2_ref[...], preferred_element_type=jnp.float32) + b2_ref[...]
        y = jnp.dot(y, w3_ref[...], preferred_element_type=jnp.float32) + b3_ref[...]
        out_ref[...] = y


def _fc_head(flat, fc1_w, fc1_b, fc2_w, fc2_b, fc3_w, fc3_b):
    ktot = fc1_w.shape[0]
    bk = 3200
    steps = ktot // bk
    return pl.pallas_call(
        _fc_body,
        grid=(steps,),
        in_specs=[
            pl.BlockSpec((1, bk), lambda k: (0, k)),
            pl.BlockSpec((bk, 256), lambda k: (k, 0)),
            pl.BlockSpec((1, 256), lambda k: (0, 0)),
            pl.BlockSpec((256, 128), lambda k: (0, 0)),
            pl.BlockSpec((1, 128), lambda k: (0, 0)),
            pl.BlockSpec((128, 2), lambda k: (0, 0)),
            pl.BlockSpec((1, 2), lambda k: (0, 0)),
        ],
        out_specs=pl.BlockSpec((1, 2), lambda k: (0, 0)),
        out_shape=jax.ShapeDtypeStruct((1, 2), jnp.float32),
        scratch_shapes=[pltpu.VMEM((1, 256), jnp.float32)],
    )(flat, fc1_w, fc1_b.reshape(1, -1), fc2_w, fc2_b.reshape(1, -1),
      fc3_w, fc3_b.reshape(1, -1))


def _rsqrt_nr(x):
    i = lax.bitcast_convert_type(x, jnp.int32)
    i = jnp.int32(0x5F3759DF) - jnp.right_shift(i, 1)
    y = lax.bitcast_convert_type(i, jnp.float32)
    for _ in range(3):
        y = y * (1.5 - 0.5 * x * y * y)
    return jnp.where(x > 0.0, y, 0.0)


def _sc_mesh():
    return plsc.VectorSubcoreMesh(core_axis_name="c", subcore_axis_name="s")


_SL = _NPAD // _NT
_NBLK = _E // (8 * _CW)


@functools.partial(
    pl.kernel,
    out_type=jax.ShapeDtypeStruct((_NBLK, 8, _CW), jnp.float32),
    mesh=_sc_mesh(),
    compiler_params=pltpu.CompilerParams(needs_layout_passes=False, use_tc_tiling_on_sc=False),
    scratch_types=[
        pltpu.VMEM_SHARED((_NPAD,), jnp.float32),
        pltpu.VMEM((8, _CW), jnp.int32),
        pltpu.VMEM((8, _CW), jnp.int32),
        pltpu.VMEM((8, _CW), jnp.float32),
        pltpu.VMEM((8, _CW), jnp.float32),
        pltpu.VMEM((_NPAD,), jnp.float32),
        pltpu.VMEM((_SL,), jnp.float32),
    ],
)
def _sc_edge_norm(row3, col3, w3, negw_hbm, deg_sh,
                  row_b, col_b, w_b, out_b, dinv_t, sl_b):
    c = lax.axis_index("c")
    s = lax.axis_index("s")
    wid = c * _NT + s
    zero = jnp.zeros((16,), jnp.float32)
    for j in range(_SL // 16):
        sl_b[pl.ds(j * 16, 16)] = zero
    pltpu.sync_copy(sl_b, deg_sh.at[pl.ds(s * _SL, _SL)])
    plsc.subcore_barrier()

    nblk_t = jnp.where(s < 8, _NBLK // _NT + 1, _NBLK // _NT)
    t0 = s * (_NBLK // _NT) + jnp.minimum(s, 8)

    def deg_chunk(i, _):
        blk = t0 + i
        pltpu.sync_copy(row3.at[blk], row_b)
        pltpu.sync_copy(w3.at[blk], w_b)
        for j in range(8):
            pltpu.sync_copy(w_b.at[j], deg_sh.at[row_b.at[j]], add=True)
        return ()

    lax.fori_loop(0, nblk_t, deg_chunk, ())
    plsc.subcore_barrier()

    pltpu.sync_copy(deg_sh.at[pl.ds(s * _SL, _SL)], sl_b)
    for j in range(_SL // 16):
        sl_b[pl.ds(j * 16, 16)] = _rsqrt_nr(sl_b[pl.ds(j * 16, 16)])
    plsc.subcore_barrier()
    pltpu.sync_copy(sl_b, deg_sh.at[pl.ds(s * _SL, _SL)])
    plsc.subcore_barrier()
    pltpu.sync_copy(deg_sh, dinv_t)

    nblk_w = jnp.where(wid < 8, _NBLK // _NW + 1, _NBLK // _NW)
    w0 = wid * (_NBLK // _NW) + jnp.minimum(wid, 8)

    def nw_chunk(i, _):
        blk = w0 + i
        pltpu.sync_copy(row3.at[blk], row_b)
        pltpu.sync_copy(col3.at[blk], col_b)
        pltpu.sync_copy(w3.at[blk], w_b)
        for j in range(8):
            for g in range(_CW // 16):
                r16 = row_b[j, pl.ds(g * 16, 16)]
                c16 = col_b[j, pl.ds(g * 16, 16)]
                wv = w_b[j, pl.ds(g * 16, 16)]
                g1 = plsc.load_gather(dinv_t, [r16])
                g2 = plsc.load_gather(dinv_t, [c16])
                out_b[j, pl.ds(g * 16, 16)] = -(g1 * wv * g2)
        pltpu.sync_copy(out_b, negw_hbm.at[blk])
        return ()

    lax.fori_loop(0, nblk_w, nw_chunk, ())


@functools.lru_cache(maxsize=None)
def _make_sc_spmm(F):
    JB = 512 // F
    NB = _E // (JB * _CW)
    q, r = divmod(NB, _NW)

    @functools.partial(
        pl.kernel,
        out_type=jax.ShapeDtypeStruct((2, _NPAD, F), jnp.float32),
        mesh=_sc_mesh(),
        compiler_params=pltpu.CompilerParams(needs_layout_passes=False, use_tc_tiling_on_sc=False),
        scratch_types=[
            pltpu.VMEM_SHARED((_NPAD, F), jnp.float32),
            pltpu.VMEM((3, JB, _CW), jnp.int32),
            pltpu.VMEM((3, JB, _CW), jnp.int32),
            pltpu.VMEM((3, JB, _CW), jnp.float32),
            pltpu.VMEM((2, JB * _CW, F), jnp.float32),
            pltpu.VMEM((16, F), jnp.float32),
            pltpu.SemaphoreType.DMA,
            pltpu.SemaphoreType.DMA,
            pltpu.SemaphoreType.DMA,
        ],
    )
    def _sc_spmm(row3, col3, negw, hmat, out_hbm, acc_sh,
                 row_b, col_b, nw_b, rows_b, z_b, sem_i, sem_g, sem_s):
        c = lax.axis_index("c")
        s = lax.axis_index("s")
        wid = c * _NT + s
        zero = jnp.zeros((16,), jnp.float32)
        for i in range(16):
            for f0 in range(0, F, 16):
                z_b[i, pl.ds(f0, 16)] = zero
        for k in range(_SL // 16):
            pltpu.async_copy(z_b, acc_sh.at[pl.ds(s * _SL + k * 16, 16)], sem_i)
        for k in range(_SL // 16):
            pltpu.make_async_copy(z_b, acc_sh.at[pl.ds(0, 16)], sem_i).wait()
        plsc.subcore_barrier()

        nb = jnp.where(wid < r, q + 1, q)
        b0 = wid * q + jnp.minimum(wid, r)
        iota = lax.iota(jnp.int32, 16)

        def idx_load(t, slot):
            blk = b0 + t
            pltpu.async_copy(row3.at[blk], row_b.at[slot], sem_i)
            pltpu.async_copy(col3.at[blk], col_b.at[slot], sem_i)
            pltpu.async_copy(negw.at[blk], nw_b.at[slot], sem_i)

        def idx_wait(slot):
            pltpu.make_async_copy(row3.at[b0], row_b.at[slot], sem_i).wait()
            pltpu.make_async_copy(col3.at[b0], col_b.at[slot], sem_i).wait()
            pltpu.make_async_copy(negw.at[b0], nw_b.at[slot], sem_i).wait()

        def gather_start(islot, rslot):
            for j in range(JB):
                pltpu.async_copy(hmat.at[row_b.at[islot].at[j]],
                                 rows_b.at[rslot].at[pl.ds(j * _CW, _CW)], sem_g)

        def gather_wait(rslot):
            for j in range(JB):
                pltpu.make_async_copy(hmat.at[pl.ds(0, _CW)],
                                      rows_b.at[rslot].at[pl.ds(j * _CW, _CW)], sem_g).wait()

        def scatter_start(islot, rslot):
            for j in range(JB):
                pltpu.async_copy(rows_b.at[rslot].at[pl.ds(j * _CW, _CW)],
                                 acc_sh.at[col_b.at[islot].at[j]], sem_s,
                                 add=True)

        def scatter_wait(rslot):
            for j in range(JB):
                pltpu.make_async_copy(rows_b.at[rslot].at[pl.ds(j * _CW, _CW)],
                                      acc_sh.at[pl.ds(0, _CW)], sem_s).wait()

        idx_load(0, 0)
        idx_wait(0)
        gather_start(0, 0)

        @pl.when(nb > 1)
        def _():
            idx_load(1, 1)

        def step(t, _):
            cur3 = lax.rem(t, 3)
            cur2 = lax.rem(t, 2)
            nxt3 = lax.rem(t + 1, 3)
            nxt2 = lax.rem(t + 1, 2)

            @pl.when(t >= 1)
            def _():
                scatter_wait(nxt2)

            @pl.when(t + 2 < nb)
            def _():
                idx_load(t + 2, lax.rem(t + 2, 3))

            gather_wait(cur2)
            def scale_j(j, _):
                for g in range(_CW // 16):
                    wv = nw_b[cur3, j, pl.ds(g * 16, 16)]
                    for l in range(16):
                        e = j * _CW + g * 16 + l
                        ws = lax.gather(
                            wv, jnp.full((16, 1), l, jnp.int32),
                            lax.GatherDimensionNumbers(
                                offset_dims=(), collapsed_slice_dims=(0,),
                                start_index_map=(0,)),
                            (1,), mode=lax.GatherScatterMode.PROMISE_IN_BOUNDS)
                        for f0 in range(0, F, 16):
                            v = rows_b[cur2, e, pl.ds(f0, 16)]
                            rows_b[cur2, e, pl.ds(f0, 16)] = v * ws
                return ()

            lax.fori_loop(0, JB, scale_j, ())
            scatter_start(cur3, cur2)

            @pl.when(t + 1 < nb)
            def _():
                idx_wait(nxt3)
                gather_start(nxt3, nxt2)
            return ()

        lax.fori_loop(0, nb, step, ())
        scatter_wait(lax.rem(nb - 1, 2))
        plsc.subcore_barrier()
        pltpu.sync_copy(acc_sh.at[pl.ds(s * _SL, _SL)],
                        out_hbm.at[c].at[pl.ds(s * _SL, _SL)])

    return _sc_spmm


def _edge_norm(row3, col3, w3):
    return _sc_edge_norm(row3, col3, w3)


def _spmm(row, col, neg_w, hmat):
    F = hmat.shape[1]
    JB = 512 // F
    nb = _E // (JB * _CW)
    r3 = row.reshape(nb, JB, _CW)
    c3 = col.reshape(nb, JB, _CW)
    n3 = neg_w.reshape(nb, JB, _CW)
    return _make_sc_spmm(F)(r3, c3, n3, hmat)


def kernel(x, edge_index, edge_weight, conv1_W0, conv1_W1, conv1_b,
           convs_W0, convs_W1, convs_b, fc1_W, fc1_b, fc2_W, fc2_b,
           fc3_W, fc3_b):
    row, col = edge_index[0], edge_index[1]
    row3 = row.reshape(_NBLK, 8, _CW)
    col3 = col.reshape(_NBLK, 8, _CW)
    w3 = edge_weight.reshape(_NBLK, 8, _CW)
    neg_w = _edge_norm(row3, col3, w3)

    t = x.shape[1]
    tpad = 64
    xp = jnp.pad(x, ((0, _NPAD - _N), (0, tpad - t)))
    w0p = jnp.pad(conv1_W0, ((0, tpad - t), (0, 0)))
    w1p = jnp.pad(conv1_W1, ((0, tpad - t), (0, 0)))
    negw_flat = neg_w.reshape(-1)
    p = _spmm(row, col, negw_flat, xp)
    h = _tc_layer(xp, p, w0p, w1p, conv1_b, True)
    for i in range(5):
        p = _spmm(row, col, negw_flat, h)
        h = _tc_layer(h, p, convs_W0[i], convs_W1[i], convs_b[i], i < 4)
    flat = h[:_N].reshape(1, -1)
    return _fc_head(flat, fc1_W, fc1_b, fc2_W, fc2_b, fc3_W, fc3_b)


# speedup vs baseline: 18.1776x; 1.0132x over previous
import functools

import jax
import jax.numpy as jnp
from jax import lax
from jax.experimental import pallas as pl
from jax.experimental.pallas import tpu as pltpu
from jax.experimental.pallas import tpu_sc as plsc

_N = 10000
_NPAD = 10240
_E = 640000
_NF = 32
_CW = 80
_NT = 16
_NW = 32


def _tc_layer_body(h_ref, p_ref, w0_ref, w1_ref, b_ref, out_ref, *, relu):
    tx1 = p_ref[0] + p_ref[1]
    m = (jnp.dot(h_ref[...], w0_ref[...], preferred_element_type=jnp.float32)
         + jnp.dot(tx1, w1_ref[...], preferred_element_type=jnp.float32)
         + b_ref[...])
    out_ref[...] = jnp.maximum(m, 0.0) if relu else m


def _tc_layer(h, p, w0, w1, b, relu):
    nf = w0.shape[1]
    return pl.pallas_call(
        functools.partial(_tc_layer_body, relu=relu),
        out_shape=jax.ShapeDtypeStruct((h.shape[0], nf), jnp.float32),
    )(h, p, w0, w1, b.reshape(1, -1))


def _fc_body(flat_ref, w1_ref, b1_ref, w2_ref, b2_ref, w3_ref, b3_ref,
             out_ref, acc_ref):
    k = pl.program_id(0)

    @pl.when(k == 0)
    def _():
        acc_ref[...] = jnp.zeros_like(acc_ref)

    acc_ref[...] += jnp.dot(flat_ref[...], w1_ref[...],
                            preferred_element_type=jnp.float32)

    @pl.when(k == pl.num_programs(0) - 1)
    def _():
        y = acc_ref[...] + b1_ref[...]
        y = jnp.dot(y, w2_ref[...], preferred_element_type=jnp.float32) + b2_ref[...]
        y = jnp.dot(y, w3_ref[...], preferred_element_type=jnp.float32) + b3_ref[...]
        out_ref[...] = y


def _fc_head(flat, fc1_w, fc1_b, fc2_w, fc2_b, fc3_w, fc3_b):
    ktot = fc1_w.shape[0]
    bk = 3200
    steps = ktot // bk
    return pl.pallas_call(
        _fc_body,
        grid=(steps,),
        in_specs=[
            pl.BlockSpec((1, bk), lambda k: (0, k)),
            pl.BlockSpec((bk, 256), lambda k: (k, 0)),
            pl.BlockSpec((1, 256), lambda k: (0, 0)),
            pl.BlockSpec((256, 128), lambda k: (0, 0)),
            pl.BlockSpec((1, 128), lambda k: (0, 0)),
            pl.BlockSpec((128, 2), lambda k: (0, 0)),
            pl.BlockSpec((1, 2), lambda k: (0, 0)),
        ],
        out_specs=pl.BlockSpec((1, 2), lambda k: (0, 0)),
        out_shape=jax.ShapeDtypeStruct((1, 2), jnp.float32),
        scratch_shapes=[pltpu.VMEM((1, 256), jnp.float32)],
    )(flat, fc1_w, fc1_b.reshape(1, -1), fc2_w, fc2_b.reshape(1, -1),
      fc3_w, fc3_b.reshape(1, -1))


def _rsqrt_nr(x):
    i = lax.bitcast_convert_type(x, jnp.int32)
    i = jnp.int32(0x5F3759DF) - jnp.right_shift(i, 1)
    y = lax.bitcast_convert_type(i, jnp.float32)
    for _ in range(3):
        y = y * (1.5 - 0.5 * x * y * y)
    return jnp.where(x > 0.0, y, 0.0)


def _sc_mesh():
    return plsc.VectorSubcoreMesh(core_axis_name="c", subcore_axis_name="s")


_SL = _NPAD // _NT
_NBLK = _E // (8 * _CW)


@functools.partial(
    pl.kernel,
    out_type=jax.ShapeDtypeStruct((_NBLK, 8, _CW), jnp.float32),
    mesh=_sc_mesh(),
    compiler_params=pltpu.CompilerParams(needs_layout_passes=False, use_tc_tiling_on_sc=False),
    scratch_types=[
        pltpu.VMEM_SHARED((_NPAD,), jnp.float32),
        pltpu.VMEM((8, _CW), jnp.int32),
        pltpu.VMEM((8, _CW), jnp.int32),
        pltpu.VMEM((8, _CW), jnp.float32),
        pltpu.VMEM((8, _CW), jnp.float32),
        pltpu.VMEM((_NPAD,), jnp.float32),
        pltpu.VMEM((_SL,), jnp.float32),
        pltpu.SemaphoreType.DMA,
    ],
)
def _sc_edge_norm(row3, col3, w3, negw_hbm, deg_sh,
                  row_b, col_b, w_b, out_b, dinv_t, sl_b, sem_d):
    c = lax.axis_index("c")
    s = lax.axis_index("s")
    wid = c * _NT + s
    zero = jnp.zeros((16,), jnp.float32)
    for j in range(_SL // 16):
        sl_b[pl.ds(j * 16, 16)] = zero
    pltpu.sync_copy(sl_b, deg_sh.at[pl.ds(s * _SL, _SL)])
    plsc.subcore_barrier()

    nblk_t = jnp.where(s < 8, _NBLK // _NT + 1, _NBLK // _NT)
    t0 = s * (_NBLK // _NT) + jnp.minimum(s, 8)

    def deg_chunk(i, _):
        blk = t0 + i
        pltpu.sync_copy(row3.at[blk], row_b)
        pltpu.sync_copy(w3.at[blk], w_b)
        for j in range(8):
            pltpu.async_copy(w_b.at[j], deg_sh.at[row_b.at[j]], sem_d,
                             add=True)
        for j in range(8):
            pltpu.make_async_copy(w_b.at[j], deg_sh.at[pl.ds(0, _CW)],
                                  sem_d).wait()
        return ()

    lax.fori_loop(0, nblk_t, deg_chunk, ())
    plsc.subcore_barrier()

    pltpu.sync_copy(deg_sh.at[pl.ds(s * _SL, _SL)], sl_b)
    for j in range(_SL // 16):
        sl_b[pl.ds(j * 16, 16)] = _rsqrt_nr(sl_b[pl.ds(j * 16, 16)])
    plsc.subcore_barrier()
    pltpu.sync_copy(sl_b, deg_sh.at[pl.ds(s * _SL, _SL)])
    plsc.subcore_barrier()
    pltpu.sync_copy(deg_sh, dinv_t)

    nblk_w = jnp.where(wid < 8, _NBLK // _NW + 1, _NBLK // _NW)
    w0 = wid * (_NBLK // _NW) + jnp.minimum(wid, 8)

    def nw_chunk(i, _):
        blk = w0 + i
        pltpu.sync_copy(row3.at[blk], row_b)
        pltpu.sync_copy(col3.at[blk], col_b)
        pltpu.sync_copy(w3.at[blk], w_b)
        for j in range(8):
            for g in range(_CW // 16):
                r16 = row_b[j, pl.ds(g * 16, 16)]
                c16 = col_b[j, pl.ds(g * 16, 16)]
                wv = w_b[j, pl.ds(g * 16, 16)]
                g1 = plsc.load_gather(dinv_t, [r16])
                g2 = plsc.load_gather(dinv_t, [c16])
                out_b[j, pl.ds(g * 16, 16)] = -(g1 * wv * g2)
        pltpu.sync_copy(out_b, negw_hbm.at[blk])
        return ()

    lax.fori_loop(0, nblk_w, nw_chunk, ())


@functools.lru_cache(maxsize=None)
def _make_sc_spmm(F):
    JB = 512 // F
    NB = _E // (JB * _CW)
    q, r = divmod(NB, _NW)

    @functools.partial(
        pl.kernel,
        out_type=jax.ShapeDtypeStruct((2, _NPAD, F), jnp.float32),
        mesh=_sc_mesh(),
        compiler_params=pltpu.CompilerParams(needs_layout_passes=False, use_tc_tiling_on_sc=False),
        scratch_types=[
            pltpu.VMEM_SHARED((_NPAD, F), jnp.float32),
            pltpu.VMEM((3, JB, _CW), jnp.int32),
            pltpu.VMEM((3, JB, _CW), jnp.int32),
            pltpu.VMEM((3, JB, _CW), jnp.float32),
            pltpu.VMEM((2, JB * _CW, F), jnp.float32),
            pltpu.VMEM((16, F), jnp.float32),
            pltpu.SemaphoreType.DMA,
            pltpu.SemaphoreType.DMA,
            pltpu.SemaphoreType.DMA,
        ],
    )
    def _sc_spmm(row3, col3, negw, hmat, out_hbm, acc_sh,
                 row_b, col_b, nw_b, rows_b, z_b, sem_i, sem_g, sem_s):
        c = lax.axis_index("c")
        s = lax.axis_index("s")
        wid = c * _NT + s
        zero = jnp.zeros((16,), jnp.float32)
        for i in range(16):
            for f0 in range(0, F, 16):
                z_b[i, pl.ds(f0, 16)] = zero
        for k in range(_SL // 16):
            pltpu.async_copy(z_b, acc_sh.at[pl.ds(s * _SL + k * 16, 16)], sem_i)
        for k in range(_SL // 16):
            pltpu.make_async_copy(z_b, acc_sh.at[pl.ds(0, 16)], sem_i).wait()
        plsc.subcore_barrier()

        nb = jnp.where(wid < r, q + 1, q)
        b0 = wid * q + jnp.minimum(wid, r)
        iota = lax.iota(jnp.int32, 16)

        def idx_load(t, slot):
            blk = b0 + t
            pltpu.async_copy(row3.at[blk], row_b.at[slot], sem_i)
            pltpu.async_copy(col3.at[blk], col_b.at[slot], sem_i)
            pltpu.async_copy(negw.at[blk], nw_b.at[slot], sem_i)

        def idx_wait(slot):
            pltpu.make_async_copy(row3.at[b0], row_b.at[slot], sem_i).wait()
            pltpu.make_async_copy(col3.at[b0], col_b.at[slot], sem_i).wait()
            pltpu.make_async_copy(negw.at[b0], nw_b.at[slot], sem_i).wait()

        def gather_start(islot, rslot):
            for j in range(JB):
                pltpu.async_copy(hmat.at[row_b.at[islot].at[j]],
                                 rows_b.at[rslot].at[pl.ds(j * _CW, _CW)], sem_g)

        def gather_wait(rslot):
            for j in range(JB):
                pltpu.make_async_copy(hmat.at[pl.ds(0, _CW)],
                                      rows_b.at[rslot].at[pl.ds(j * _CW, _CW)], sem_g).wait()

        def scatter_start(islot, rslot):
            for j in range(JB):
                pltpu.async_copy(rows_b.at[rslot].at[pl.ds(j * _CW, _CW)],
                                 acc_sh.at[col_b.at[islot].at[j]], sem_s,
                                 add=True)

        def scatter_wait(rslot):
            for j in range(JB):
                pltpu.make_async_copy(rows_b.at[rslot].at[pl.ds(j * _CW, _CW)],
                                      acc_sh.at[pl.ds(0, _CW)], sem_s).wait()

        idx_load(0, 0)
        idx_wait(0)
        gather_start(0, 0)

        @pl.when(nb > 1)
        def _():
            idx_load(1, 1)

        def step(t, _):
            cur3 = lax.rem(t, 3)
            cur2 = lax.rem(t, 2)
            nxt3 = lax.rem(t + 1, 3)
            nxt2 = lax.rem(t + 1, 2)

            @pl.when(t >= 1)
            def _():
                scatter_wait(nxt2)

            @pl.when(t + 2 < nb)
            def _():
                idx_load(t + 2, lax.rem(t + 2, 3))

            gather_wait(cur2)
            def scale_j(j, _):
                for g in range(_CW // 16):
                    wv = nw_b[cur3, j, pl.ds(g * 16, 16)]
                    for l in range(16):
                        e = j * _CW + g * 16 + l
                        ws = lax.gather(
                            wv, jnp.full((16, 1), l, jnp.int32),
                            lax.GatherDimensionNumbers(
                                offset_dims=(), collapsed_slice_dims=(0,),
                                start_index_map=(0,)),
                            (1,), mode=lax.GatherScatterMode.PROMISE_IN_BOUNDS)
                        for f0 in range(0, F, 16):
                            v = rows_b[cur2, e, pl.ds(f0, 16)]
                            rows_b[cur2, e, pl.ds(f0, 16)] = v * ws
                return ()

            lax.fori_loop(0, JB, scale_j, ())
            scatter_start(cur3, cur2)

            @pl.when(t + 1 < nb)
            def _():
                idx_wait(nxt3)
                gather_start(nxt3, nxt2)
            return ()

        lax.fori_loop(0, nb, step, ())
        scatter_wait(lax.rem(nb - 1, 2))
        plsc.subcore_barrier()
        pltpu.sync_copy(acc_sh.at[pl.ds(s * _SL, _SL)],
                        out_hbm.at[c].at[pl.ds(s * _SL, _SL)])

    return _sc_spmm


def _edge_norm(row3, col3, w3):
    return _sc_edge_norm(row3, col3, w3)


def _spmm(row, col, neg_w, hmat):
    F = hmat.shape[1]
    JB = 512 // F
    nb = _E // (JB * _CW)
    r3 = row.reshape(nb, JB, _CW)
    c3 = col.reshape(nb, JB, _CW)
    n3 = neg_w.reshape(nb, JB, _CW)
    return _make_sc_spmm(F)(r3, c3, n3, hmat)


def kernel(x, edge_index, edge_weight, conv1_W0, conv1_W1, conv1_b,
           convs_W0, convs_W1, convs_b, fc1_W, fc1_b, fc2_W, fc2_b,
           fc3_W, fc3_b):
    row, col = edge_index[0], edge_index[1]
    row3 = row.reshape(_NBLK, 8, _CW)
    col3 = col.reshape(_NBLK, 8, _CW)
    w3 = edge_weight.reshape(_NBLK, 8, _CW)
    neg_w = _edge_norm(row3, col3, w3)

    t = x.shape[1]
    tpad = 64
    xp = jnp.pad(x, ((0, _NPAD - _N), (0, tpad - t)))
    w0p = jnp.pad(conv1_W0, ((0, tpad - t), (0, 0)))
    w1p = jnp.pad(conv1_W1, ((0, tpad - t), (0, 0)))
    negw_flat = neg_w.reshape(-1)
    p = _spmm(row, col, negw_flat, xp)
    h = _tc_layer(xp, p, w0p, w1p, conv1_b, True)
    for i in range(5):
        p = _spmm(row, col, negw_flat, h)
        h = _tc_layer(h, p, convs_W0[i], convs_W1[i], convs_b[i], i < 4)
    flat = h[:_N].reshape(1, -1)
    return _fc_head(flat, fc1_W, fc1_b, fc2_W, fc2_b, fc3_W, fc3_b)


# speedup vs baseline: 18.1892x; 1.0006x over previous
import functools

import jax
import jax.numpy as jnp
from jax import lax
from jax.experimental import pallas as pl
from jax.experimental.pallas import tpu as pltpu
from jax.experimental.pallas import tpu_sc as plsc

_N = 10000
_NPAD = 10240
_E = 640000
_NF = 32
_CW = 80
_NT = 16
_NW = 32


def _tc_layer_body(h_ref, p_ref, w0_ref, w1_ref, b_ref, out_ref, *, relu):
    tx1 = p_ref[0] + p_ref[1]
    m = (jnp.dot(h_ref[...], w0_ref[...], preferred_element_type=jnp.float32)
         + jnp.dot(tx1, w1_ref[...], preferred_element_type=jnp.float32)
         + b_ref[...])
    out_ref[...] = jnp.maximum(m, 0.0) if relu else m


def _tc_layer(h, p, w0, w1, b, relu):
    nf = w0.shape[1]
    return pl.pallas_call(
        functools.partial(_tc_layer_body, relu=relu),
        out_shape=jax.ShapeDtypeStruct((h.shape[0], nf), jnp.float32),
    )(h, p, w0, w1, b.reshape(1, -1))


def _fc_body(flat_ref, w1_ref, b1_ref, w2_ref, b2_ref, w3_ref, b3_ref,
             out_ref, acc_ref):
    k = pl.program_id(0)

    @pl.when(k == 0)
    def _():
        acc_ref[...] = jnp.zeros_like(acc_ref)

    acc_ref[...] += jnp.dot(flat_ref[...], w1_ref[...],
                            preferred_element_type=jnp.float32)

    @pl.when(k == pl.num_programs(0) - 1)
    def _():
        y = acc_ref[...] + b1_ref[...]
        y = jnp.dot(y, w2_ref[...], preferred_element_type=jnp.float32) + b2_ref[...]
        y = jnp.dot(y, w3_ref[...], preferred_element_type=jnp.float32) + b3_ref[...]
        out_ref[...] = y


def _fc_head(flat, fc1_w, fc1_b, fc2_w, fc2_b, fc3_w, fc3_b):
    ktot = fc1_w.shape[0]
    bk = 3200
    steps = ktot // bk
    return pl.pallas_call(
        _fc_body,
        grid=(steps,),
        in_specs=[
            pl.BlockSpec((1, bk), lambda k: (0, k)),
            pl.BlockSpec((bk, 256), lambda k: (k, 0)),
            pl.BlockSpec((1, 256), lambda k: (0, 0)),
            pl.BlockSpec((256, 128), lambda k: (0, 0)),
            pl.BlockSpec((1, 128), lambda k: (0, 0)),
            pl.BlockSpec((128, 2), lambda k: (0, 0)),
            pl.BlockSpec((1, 2), lambda k: (0, 0)),
        ],
        out_specs=pl.BlockSpec((1, 2), lambda k: (0, 0)),
        out_shape=jax.ShapeDtypeStruct((1, 2), jnp.float32),
        scratch_shapes=[pltpu.VMEM((1, 256), jnp.float32)],
    )(flat, fc1_w, fc1_b.reshape(1, -1), fc2_w, fc2_b.reshape(1, -1),
      fc3_w, fc3_b.reshape(1, -1))


def _rsqrt_nr(x):
    i = lax.bitcast_convert_type(x, jnp.int32)
    i = jnp.int32(0x5F3759DF) - jnp.right_shift(i, 1)
    y = lax.bitcast_convert_type(i, jnp.float32)
    for _ in range(3):
        y = y * (1.5 - 0.5 * x * y * y)
    return jnp.where(x > 0.0, y, 0.0)


def _sc_mesh():
    return plsc.VectorSubcoreMesh(core_axis_name="c", subcore_axis_name="s")


_SL = _NPAD // _NT
_NBLK = _E // (8 * _CW)


@functools.partial(
    pl.kernel,
    out_type=jax.ShapeDtypeStruct((_NBLK, 8, _CW), jnp.float32),
    mesh=_sc_mesh(),
    compiler_params=pltpu.CompilerParams(needs_layout_passes=False, use_tc_tiling_on_sc=False),
    scratch_types=[
        pltpu.VMEM_SHARED((_NPAD,), jnp.float32),
        pltpu.VMEM((8, _CW), jnp.int32),
        pltpu.VMEM((8, _CW), jnp.int32),
        pltpu.VMEM((8, _CW), jnp.float32),
        pltpu.VMEM((8, _CW), jnp.float32),
        pltpu.VMEM((_NPAD,), jnp.float32),
        pltpu.VMEM((_SL,), jnp.float32),
        pltpu.SemaphoreType.DMA,
    ],
)
def _sc_edge_norm(row3, col3, w3, negw_hbm, deg_sh,
                  row_b, col_b, w_b, out_b, dinv_t, sl_b, sem_d):
    c = lax.axis_index("c")
    s = lax.axis_index("s")
    wid = c * _NT + s
    zero = jnp.zeros((16,), jnp.float32)
    for j in range(_SL // 16):
        sl_b[pl.ds(j * 16, 16)] = zero
    pltpu.sync_copy(sl_b, deg_sh.at[pl.ds(s * _SL, _SL)])
    plsc.subcore_barrier()

    nblk_t = jnp.where(s < 8, _NBLK // _NT + 1, _NBLK // _NT)
    t0 = s * (_NBLK // _NT) + jnp.minimum(s, 8)

    def deg_chunk(i, _):
        blk = t0 + i
        pltpu.sync_copy(row3.at[blk], row_b)
        pltpu.sync_copy(w3.at[blk], w_b)
        for j in range(8):
            pltpu.async_copy(w_b.at[j], deg_sh.at[row_b.at[j]], sem_d,
                             add=True)
        for j in range(8):
            pltpu.make_async_copy(w_b.at[j], deg_sh.at[pl.ds(0, _CW)],
                                  sem_d).wait()
        return ()

    lax.fori_loop(0, nblk_t, deg_chunk, ())
    plsc.subcore_barrier()

    pltpu.sync_copy(deg_sh.at[pl.ds(s * _SL, _SL)], sl_b)
    for j in range(_SL // 16):
        sl_b[pl.ds(j * 16, 16)] = _rsqrt_nr(sl_b[pl.ds(j * 16, 16)])
    plsc.subcore_barrier()
    pltpu.sync_copy(sl_b, deg_sh.at[pl.ds(s * _SL, _SL)])
    plsc.subcore_barrier()
    pltpu.sync_copy(deg_sh, dinv_t)

    nblk_w = jnp.where(wid < 8, _NBLK // _NW + 1, _NBLK // _NW)
    w0 = wid * (_NBLK // _NW) + jnp.minimum(wid, 8)

    def nw_chunk(i, _):
        blk = w0 + i
        pltpu.sync_copy(row3.at[blk], row_b)
        pltpu.sync_copy(col3.at[blk], col_b)
        pltpu.sync_copy(w3.at[blk], w_b)
        for j in range(8):
            for g in range(_CW // 16):
                r16 = row_b[j, pl.ds(g * 16, 16)]
                c16 = col_b[j, pl.ds(g * 16, 16)]
                wv = w_b[j, pl.ds(g * 16, 16)]
                g1 = plsc.load_gather(dinv_t, [r16])
                g2 = plsc.load_gather(dinv_t, [c16])
                out_b[j, pl.ds(g * 16, 16)] = -(g1 * wv * g2)
        pltpu.sync_copy(out_b, negw_hbm.at[blk])
        return ()

    lax.fori_loop(0, nblk_w, nw_chunk, ())


@functools.lru_cache(maxsize=None)
def _make_sc_spmm(F):
    JB = 512 // F
    NB = _E // (JB * _CW)
    q, r = divmod(NB, _NW)

    @functools.partial(
        pl.kernel,
        out_type=jax.ShapeDtypeStruct((2, _NPAD, F), jnp.float32),
        mesh=_sc_mesh(),
        compiler_params=pltpu.CompilerParams(needs_layout_passes=False, use_tc_tiling_on_sc=False),
        scratch_types=[
            pltpu.VMEM_SHARED((_NPAD, F), jnp.float32),
            pltpu.VMEM((3, JB, _CW), jnp.int32),
            pltpu.VMEM((3, JB, _CW), jnp.int32),
            pltpu.VMEM((3, JB, _CW), jnp.float32),
            pltpu.VMEM((2, JB * _CW, F), jnp.float32),
            pltpu.VMEM((16, F), jnp.float32),
            pltpu.SemaphoreType.DMA,
            pltpu.SemaphoreType.DMA,
            pltpu.SemaphoreType.DMA,
        ],
    )
    def _sc_spmm(row3, col3, negw, hmat, out_hbm, acc_sh,
                 row_b, col_b, nw_b, rows_b, z_b, sem_i, sem_g, sem_s):
        c = lax.axis_index("c")
        s = lax.axis_index("s")
        wid = c * _NT + s
        zero = jnp.zeros((16,), jnp.float32)
        for i in range(16):
            for f0 in range(0, F, 16):
                z_b[i, pl.ds(f0, 16)] = zero
        for k in range(_SL // 16):
            pltpu.async_copy(z_b, acc_sh.at[pl.ds(s * _SL + k * 16, 16)], sem_i)
        for k in range(_SL // 16):
            pltpu.make_async_copy(z_b, acc_sh.at[pl.ds(0, 16)], sem_i).wait()
        plsc.subcore_barrier()

        nb = jnp.where(wid < r, q + 1, q)
        b0 = wid * q + jnp.minimum(wid, r)

        def idx_load(t, slot):
            blk = b0 + t
            pltpu.async_copy(row3.at[blk], row_b.at[slot], sem_i)
            pltpu.async_copy(col3.at[blk], col_b.at[slot], sem_i)
            pltpu.async_copy(negw.at[blk], nw_b.at[slot], sem_i)

        def idx_wait(slot):
            pltpu.make_async_copy(row3.at[b0], row_b.at[slot], sem_i).wait()
            pltpu.make_async_copy(col3.at[b0], col_b.at[slot], sem_i).wait()
            pltpu.make_async_copy(negw.at[b0], nw_b.at[slot], sem_i).wait()

        def gather_start(islot, rslot):
            for j in range(JB):
                pltpu.async_copy(hmat.at[row_b.at[islot].at[j]],
                                 rows_b.at[rslot].at[pl.ds(j * _CW, _CW)], sem_g)

        def gather_wait(rslot):
            for j in range(JB):
                pltpu.make_async_copy(hmat.at[pl.ds(0, _CW)],
                                      rows_b.at[rslot].at[pl.ds(j * _CW, _CW)], sem_g).wait()

        def scatter_start(islot, rslot):
            for j in range(JB):
                pltpu.async_copy(rows_b.at[rslot].at[pl.ds(j * _CW, _CW)],
                                 acc_sh.at[col_b.at[islot].at[j]], sem_s,
                                 add=True)

        def scatter_wait(rslot):
            for j in range(JB):
                pltpu.make_async_copy(rows_b.at[rslot].at[pl.ds(j * _CW, _CW)],
                                      acc_sh.at[pl.ds(0, _CW)], sem_s).wait()

        idx_load(0, 0)
        idx_wait(0)
        gather_start(0, 0)

        @pl.when(nb > 1)
        def _():
            idx_load(1, 1)

        def step(t, _):
            cur3 = lax.rem(t, 3)
            cur2 = lax.rem(t, 2)
            nxt3 = lax.rem(t + 1, 3)
            nxt2 = lax.rem(t + 1, 2)

            @pl.when(t >= 1)
            def _():
                scatter_wait(nxt2)

            @pl.when(t + 2 < nb)
            def _():
                idx_load(t + 2, lax.rem(t + 2, 3))

            gather_wait(cur2)
            def scale_j(j, _):
                for g in range(_CW // 16):
                    wv = nw_b[cur3, j, pl.ds(g * 16, 16)]
                    for l in range(16):
                        e = j * _CW + g * 16 + l
                        ws = lax.gather(
                            wv, jnp.full((16, 1), l, jnp.int32),
                            lax.GatherDimensionNumbers(
                                offset_dims=(), collapsed_slice_dims=(0,),
                                start_index_map=(0,)),
                            (1,), mode=lax.GatherScatterMode.PROMISE_IN_BOUNDS)
                        for f0 in range(0, F, 16):
                            v = rows_b[cur2, e, pl.ds(f0, 16)]
                            rows_b[cur2, e, pl.ds(f0, 16)] = v * ws
                return ()

            lax.fori_loop(0, JB, scale_j, ())
            scatter_start(cur3, cur2)

            @pl.when(t + 1 < nb)
            def _():
                idx_wait(nxt3)
                gather_start(nxt3, nxt2)
            return ()

        lax.fori_loop(0, nb, step, ())
        scatter_wait(lax.rem(nb - 1, 2))
        plsc.subcore_barrier()
        pltpu.sync_copy(acc_sh.at[pl.ds(s * _SL, _SL)],
                        out_hbm.at[c].at[pl.ds(s * _SL, _SL)])

    return _sc_spmm


def _edge_norm(row3, col3, w3):
    return _sc_edge_norm(row3, col3, w3)


def _spmm(row, col, neg_w, hmat):
    F = hmat.shape[1]
    JB = 512 // F
    nb = _E // (JB * _CW)
    r3 = row.reshape(nb, JB, _CW)
    c3 = col.reshape(nb, JB, _CW)
    n3 = neg_w.reshape(nb, JB, _CW)
    return _make_sc_spmm(F)(r3, c3, n3, hmat)


def kernel(x, edge_index, edge_weight, conv1_W0, conv1_W1, conv1_b,
           convs_W0, convs_W1, convs_b, fc1_W, fc1_b, fc2_W, fc2_b,
           fc3_W, fc3_b):
    row, col = edge_index[0], edge_index[1]
    row3 = row.reshape(_NBLK, 8, _CW)
    col3 = col.reshape(_NBLK, 8, _CW)
    w3 = edge_weight.reshape(_NBLK, 8, _CW)
    neg_w = _edge_norm(row3, col3, w3)

    t = x.shape[1]
    tpad = 64
    xp = jnp.pad(x, ((0, _NPAD - _N), (0, tpad - t)))
    w0p = jnp.pad(conv1_W0, ((0, tpad - t), (0, 0)))
    w1p = jnp.pad(conv1_W1, ((0, tpad - t), (0, 0)))
    negw_flat = neg_w.reshape(-1)
    p = _spmm(row, col, negw_flat, xp)
    h = _tc_layer(xp, p, w0p, w1p, conv1_b, True)
    for i in range(5):
        p = _spmm(row, col, negw_flat, h)
        h = _tc_layer(h, p, convs_W0[i], convs_W1[i], convs_b[i], i < 4)
    flat = h[:_N].reshape(1, -1)
    return _fc_head(flat, fc1_W, fc1_b, fc2_W, fc2_b, fc3_W, fc3_b)
